# baseline (speedup 1.0000x reference)
import functools

import jax
import jax.numpy as jnp
from jax import lax
from jax.experimental import pallas as pl
from jax.experimental.pallas import tpu as pltpu
from jax.experimental.pallas import tpu_sc as plsc

N_LAT = 32
LANES = 16
IDX_CHUNK = 128


def _mf_sc_call(R, n_users, n_items):
    info = plsc.get_sparse_core_info()
    NC, NS = info.num_cores, info.num_subcores
    NW = NC * NS
    RW = R // NW
    NCH = RW // IDX_CHUNK
    NG = RW // LANES

    mesh = plsc.VectorSubcoreMesh(core_axis_name="c", subcore_axis_name="s")

    @functools.partial(
        pl.kernel,
        out_type=[
            jax.ShapeDtypeStruct((R,), jnp.float32),
            jax.ShapeDtypeStruct((NC, LANES), jnp.float32),
        ],
        mesh=mesh,
        compiler_params=pltpu.CompilerParams(
            needs_layout_passes=False, use_tc_tiling_on_sc=False),
        scratch_types=[
            pltpu.VMEM((NCH, IDX_CHUNK), jnp.int32),
            pltpu.VMEM((NCH, IDX_CHUNK), jnp.int32),
            pltpu.VMEM((RW, N_LAT), jnp.float32),
            pltpu.VMEM((RW, N_LAT), jnp.float32),
            pltpu.VMEM((RW,), jnp.float32),
            pltpu.VMEM((RW,), jnp.float32),
            pltpu.VMEM((LANES,), jnp.float32),
            pltpu.VMEM((NS, LANES), jnp.float32),
            pltpu.VMEM((LANES,), jnp.float32),
            pltpu.VMEM_SHARED((NS, LANES), jnp.float32),
            pltpu.SemaphoreType.DMA,
        ],
    )
    def mf(theta_h, x_h, ui_h, ii_h, rat_h, pred_h, loss_h,
           ui_v, ii_v, u_rows, x_rows, rat_v, pred_v,
           sq_v, accv, lossv, shared, sem):
        cid = lax.axis_index("c")
        sid = lax.axis_index("s")
        wid = sid * NC + cid
        base = wid * RW

        for j in range(NCH):
            pltpu.sync_copy(ui_h.at[pl.ds(base + j * IDX_CHUNK, IDX_CHUNK)], ui_v.at[j])
            pltpu.sync_copy(ii_h.at[pl.ds(base + j * IDX_CHUNK, IDX_CHUNK)], ii_v.at[j])
        pltpu.sync_copy(rat_h.at[pl.ds(base, RW)], rat_v)

        cps = []
        for j in range(NCH):
            cps.append(pltpu.async_copy(
                theta_h.at[ui_v.at[j]], u_rows.at[pl.ds(j * IDX_CHUNK, IDX_CHUNK)], sem))
            cps.append(pltpu.async_copy(
                x_h.at[ii_v.at[j]], x_rows.at[pl.ds(j * IDX_CHUNK, IDX_CHUNK)], sem))
        for cp in cps:
            cp.wait()

        iota = lax.iota(jnp.int32, LANES)

        def group_body(g, sq):
            row0 = g * LANES
            svec = jnp.zeros((LANES,), jnp.float32)
            for r in range(LANES):
                u0 = u_rows[row0 + r, pl.ds(0, LANES)]
                u1 = u_rows[row0 + r, pl.ds(LANES, LANES)]
                x0 = x_rows[row0 + r, pl.ds(0, LANES)]
                x1 = x_rows[row0 + r, pl.ds(LANES, LANES)]
                w = u0 * x0 + u1 * x1
                s = jnp.sum(w)
                svec = jnp.where(iota == r, s, svec)
            pred_v[pl.ds(row0, LANES)] = svec
            d = svec - rat_v[pl.ds(row0, LANES)]
            return sq + d * d

        sq = lax.fori_loop(0, NG, group_body, jnp.zeros((LANES,), jnp.float32))
        sq_v[...] = sq

        pltpu.sync_copy(pred_v, pred_h.at[pl.ds(base, RW)])

        pltpu.sync_copy(sq_v, shared.at[sid])
        plsc.subcore_barrier()

        @pl.when(sid == 0)
        def _():
            pltpu.sync_copy(shared, accv)
            tot = accv[0]
            for i in range(1, NS):
                tot = tot + accv[i]
            lossv[...] = tot
            pltpu.sync_copy(lossv, loss_h.at[cid])

    return mf


def kernel(theta, X, user_indices, item_indices, ratings):
    R = user_indices.shape[0]
    mf = _mf_sc_call(R, theta.shape[0], X.shape[0])
    pred, loss_parts = mf(
        theta, X,
        user_indices.astype(jnp.int32),
        item_indices.astype(jnp.int32),
        ratings,
    )
    loss = jnp.sum(loss_parts) / R
    return pred, loss


# speedup vs baseline: 1.6346x; 1.6346x over previous
import functools

import jax
import jax.numpy as jnp
from jax import lax
from jax.experimental import pallas as pl
from jax.experimental.pallas import tpu as pltpu
from jax.experimental.pallas import tpu_sc as plsc

N_LAT = 32
LANES = 16
SR = 128
IDX_CHUNK = 128
TC_BLK = 2048


def _repack(table, k_stride):
    table_t = table.T
    nb = k_stride // TC_BLK
    last = (table.shape[0] - 1) // TC_BLK

    def body(x0, x1, x2, x3, out_ref):
        parts = [jnp.transpose(x[...]) for x in (x0, x1, x2, x3)]
        out_ref[...] = jnp.concatenate(parts, axis=1)

    in_specs = [
        pl.BlockSpec(
            (N_LAT, TC_BLK),
            functools.partial(lambda k, b: (0, jnp.minimum(nb * k + b, last)), k),
        )
        for k in range(4)
    ]
    return pl.pallas_call(
        body,
        grid=(nb,),
        in_specs=in_specs,
        out_specs=pl.BlockSpec((TC_BLK, SR), lambda b: (b, 0)),
        out_shape=jax.ShapeDtypeStruct((k_stride, SR), jnp.float32),
    )(table_t, table_t, table_t, table_t)


def _mf_sc_call(R, ku, kx):
    info = plsc.get_sparse_core_info()
    NC, NS = info.num_cores, info.num_subcores
    NW = NC * NS
    RW = R // NW
    NCH = RW // IDX_CHUNK
    HALF = RW // 2
    NGH = HALF // LANES
    ushift = ku.bit_length() - 1
    xshift = kx.bit_length() - 1

    mesh = plsc.VectorSubcoreMesh(core_axis_name="c", subcore_axis_name="s")

    @functools.partial(
        pl.kernel,
        out_type=[
            jax.ShapeDtypeStruct((R,), jnp.float32),
            jax.ShapeDtypeStruct((NC, LANES), jnp.float32),
        ],
        mesh=mesh,
        compiler_params=pltpu.CompilerParams(
            needs_layout_passes=False, use_tc_tiling_on_sc=False),
        scratch_types=[
            pltpu.VMEM((NCH, IDX_CHUNK), jnp.int32),
            pltpu.VMEM((NCH, IDX_CHUNK), jnp.int32),
            pltpu.VMEM((NCH, IDX_CHUNK), jnp.int32),
            pltpu.VMEM((NCH, IDX_CHUNK), jnp.int32),
            pltpu.VMEM((HALF, SR), jnp.float32),
            pltpu.VMEM((HALF, SR), jnp.float32),
            pltpu.VMEM((RW,), jnp.float32),
            pltpu.VMEM((RW,), jnp.float32),
            pltpu.VMEM((LANES,), jnp.float32),
            pltpu.VMEM((NS, LANES), jnp.float32),
            pltpu.VMEM((LANES,), jnp.float32),
            pltpu.VMEM_SHARED((NS, LANES), jnp.float32),
            pltpu.SemaphoreType.DMA,
        ],
    )
    def mf(theta_h, x_h, ui_h, ii_h, rat_h, pred_h, loss_h,
           ui_v, ii_v, sui_v, sii_v, u_rows, x_rows, rat_v, pred_v,
           sq_v, accv, lossv, shared, sem):
        cid = lax.axis_index("c")
        sid = lax.axis_index("s")
        wid = sid * NC + cid
        base = wid * RW

        for j in range(NCH):
            pltpu.sync_copy(ui_h.at[pl.ds(base + j * IDX_CHUNK, IDX_CHUNK)], ui_v.at[j])
            pltpu.sync_copy(ii_h.at[pl.ds(base + j * IDX_CHUNK, IDX_CHUNK)], ii_v.at[j])
        pltpu.sync_copy(rat_h.at[pl.ds(base, RW)], rat_v)

        for j in range(NCH):
            for k in range(IDX_CHUNK // LANES):
                sl = pl.ds(k * LANES, LANES)
                sui_v[j, sl] = ui_v[j, sl] & (ku - 1)
                sii_v[j, sl] = ii_v[j, sl] & (kx - 1)

        iota = lax.iota(jnp.int32, LANES)

        for half in range(2):
            cps = []
            for j in range(2):
                ch = 2 * half + j
                cps.append(pltpu.async_copy(
                    theta_h.at[sui_v.at[ch]],
                    u_rows.at[pl.ds(j * IDX_CHUNK, IDX_CHUNK)], sem))
                cps.append(pltpu.async_copy(
                    x_h.at[sii_v.at[ch]],
                    x_rows.at[pl.ds(j * IDX_CHUNK, IDX_CHUNK)], sem))
            for cp in cps:
                cp.wait()

            def group_body(g, sq):
                lrow0 = g * LANES
                row0 = half * HALF + lrow0
                ch = 2 * half + lrow0 // IDX_CHUNK
                col0 = lrow0 % IDX_CHUNK
                uo_vec = lax.shift_right_logical(
                    ui_v[ch, pl.ds(col0, LANES)], ushift) * N_LAT
                xo_vec = lax.shift_right_logical(
                    ii_v[ch, pl.ds(col0, LANES)], xshift) * N_LAT
                svec = jnp.zeros((LANES,), jnp.float32)
                for r in range(LANES):
                    uo = uo_vec[r]
                    xo = xo_vec[r]
                    u0 = u_rows[lrow0 + r, pl.ds(uo, LANES)]
                    u1 = u_rows[lrow0 + r, pl.ds(uo + LANES, LANES)]
                    x0 = x_rows[lrow0 + r, pl.ds(xo, LANES)]
                    x1 = x_rows[lrow0 + r, pl.ds(xo + LANES, LANES)]
                    w = u0 * x0 + u1 * x1
                    s = jnp.sum(w)
                    svec = jnp.where(iota == r, s, svec)
                pred_v[pl.ds(row0, LANES)] = svec
                d = svec - rat_v[pl.ds(row0, LANES)]
                return sq + d * d

            init = sq_v[...] if half else jnp.zeros((LANES,), jnp.float32)
            sq_v[...] = lax.fori_loop(0, NGH, group_body, init)

        pltpu.sync_copy(pred_v, pred_h.at[pl.ds(base, RW)])

        pltpu.sync_copy(sq_v, shared.at[sid])
        plsc.subcore_barrier()

        @pl.when(sid == 0)
        def _():
            pltpu.sync_copy(shared, accv)
            tot = accv[0]
            for i in range(1, NS):
                tot = tot + accv[i]
            lossv[...] = tot
            pltpu.sync_copy(lossv, loss_h.at[cid])

    return mf


def kernel(theta, X, user_indices, item_indices, ratings):
    R = user_indices.shape[0]
    ku = max(1 << (theta.shape[0] - 1).bit_length(), 4 * TC_BLK) // 4
    kx = max(1 << (X.shape[0] - 1).bit_length(), 4 * TC_BLK) // 4
    theta_q = _repack(theta, ku)
    x_q = _repack(X, kx)
    mf = _mf_sc_call(R, ku, kx)
    pred, loss_parts = mf(
        theta_q, x_q,
        user_indices.astype(jnp.int32),
        item_indices.astype(jnp.int32),
        ratings,
    )
    loss = jnp.sum(loss_parts) / R
    return pred, loss


# speedup vs baseline: 2.8116x; 1.7200x over previous
import functools

import jax
import jax.numpy as jnp
from jax import lax
from jax.experimental import pallas as pl
from jax.experimental.pallas import tpu as pltpu
from jax.experimental.pallas import tpu_sc as plsc

N_LAT = 32
LANES = 16
SR = 128
IDX_CHUNK = 128
TC_BLK = 2048


def _repack(table, k_stride):
    table_t = table.T
    nb = k_stride // TC_BLK
    last = (table.shape[0] - 1) // TC_BLK

    def body(x0, x1, x2, x3, out_ref):
        x_cat = jnp.concatenate([x[...] for x in (x0, x1, x2, x3)], axis=0)
        out_ref[...] = jnp.transpose(x_cat)

    in_specs = [
        pl.BlockSpec(
            (N_LAT, TC_BLK),
            functools.partial(lambda k, b: (0, jnp.minimum(nb * k + b, last)), k),
        )
        for k in range(4)
    ]
    return pl.pallas_call(
        body,
        grid=(nb,),
        in_specs=in_specs,
        out_specs=pl.BlockSpec((TC_BLK, SR), lambda b: (b, 0)),
        out_shape=jax.ShapeDtypeStruct((k_stride, SR), jnp.float32),
    )(table_t, table_t, table_t, table_t)


def _mf_sc_call(R, ku, kx):
    info = plsc.get_sparse_core_info()
    NC, NS = info.num_cores, info.num_subcores
    NW = NC * NS
    RW = R // NW
    NCH = RW // IDX_CHUNK
    HALF = RW // 2
    NGH = HALF // LANES
    ushift = ku.bit_length() - 1
    xshift = kx.bit_length() - 1

    mesh = plsc.VectorSubcoreMesh(core_axis_name="c", subcore_axis_name="s")

    @functools.partial(
        pl.kernel,
        out_type=[
            jax.ShapeDtypeStruct((R,), jnp.float32),
            jax.ShapeDtypeStruct((NC, LANES), jnp.float32),
        ],
        mesh=mesh,
        compiler_params=pltpu.CompilerParams(
            needs_layout_passes=False, use_tc_tiling_on_sc=False),
        scratch_types=[
            pltpu.VMEM((NCH, IDX_CHUNK), jnp.int32),
            pltpu.VMEM((NCH, IDX_CHUNK), jnp.int32),
            pltpu.VMEM((NCH, IDX_CHUNK), jnp.int32),
            pltpu.VMEM((NCH, IDX_CHUNK), jnp.int32),
            pltpu.VMEM((HALF, SR), jnp.float32),
            pltpu.VMEM((HALF, SR), jnp.float32),
            pltpu.VMEM((RW,), jnp.float32),
            pltpu.VMEM((RW,), jnp.float32),
            pltpu.VMEM((LANES,), jnp.float32),
            pltpu.VMEM((NS, LANES), jnp.float32),
            pltpu.VMEM((LANES,), jnp.float32),
            pltpu.VMEM_SHARED((NS, LANES), jnp.float32),
            pltpu.SemaphoreType.DMA,
        ],
    )
    def mf(theta_h, x_h, ui_h, ii_h, rat_h, pred_h, loss_h,
           ui_v, ii_v, sui_v, sii_v, u_rows, x_rows, rat_v, pred_v,
           sq_v, accv, lossv, shared, sem):
        cid = lax.axis_index("c")
        sid = lax.axis_index("s")
        wid = sid * NC + cid
        base = wid * RW

        for j in range(NCH):
            pltpu.sync_copy(ui_h.at[pl.ds(base + j * IDX_CHUNK, IDX_CHUNK)], ui_v.at[j])
            pltpu.sync_copy(ii_h.at[pl.ds(base + j * IDX_CHUNK, IDX_CHUNK)], ii_v.at[j])
        pltpu.sync_copy(rat_h.at[pl.ds(base, RW)], rat_v)

        for j in range(NCH):
            for k in range(IDX_CHUNK // LANES):
                sl = pl.ds(k * LANES, LANES)
                sui_v[j, sl] = ui_v[j, sl] & (ku - 1)
                sii_v[j, sl] = ii_v[j, sl] & (kx - 1)

        iota = lax.iota(jnp.int32, LANES)

        for half in range(2):
            cps = []
            for j in range(2):
                ch = 2 * half + j
                cps.append(pltpu.async_copy(
                    theta_h.at[sui_v.at[ch]],
                    u_rows.at[pl.ds(j * IDX_CHUNK, IDX_CHUNK)], sem))
                cps.append(pltpu.async_copy(
                    x_h.at[sii_v.at[ch]],
                    x_rows.at[pl.ds(j * IDX_CHUNK, IDX_CHUNK)], sem))
            for cp in cps:
                cp.wait()

            def group_body(g, sq):
                lrow0 = g * LANES
                row0 = half * HALF + lrow0
                ch = 2 * half + lrow0 // IDX_CHUNK
                col0 = lrow0 % IDX_CHUNK
                uo_vec = lax.shift_right_logical(
                    ui_v[ch, pl.ds(col0, LANES)], ushift) * N_LAT
                xo_vec = lax.shift_right_logical(
                    ii_v[ch, pl.ds(col0, LANES)], xshift) * N_LAT
                svec = jnp.zeros((LANES,), jnp.float32)
                for r in range(LANES):
                    uo = uo_vec[r]
                    xo = xo_vec[r]
                    u0 = u_rows[lrow0 + r, pl.ds(uo, LANES)]
                    u1 = u_rows[lrow0 + r, pl.ds(uo + LANES, LANES)]
                    x0 = x_rows[lrow0 + r, pl.ds(xo, LANES)]
                    x1 = x_rows[lrow0 + r, pl.ds(xo + LANES, LANES)]
                    w = u0 * x0 + u1 * x1
                    s = jnp.sum(w)
                    svec = jnp.where(iota == r, s, svec)
                pred_v[pl.ds(row0, LANES)] = svec
                d = svec - rat_v[pl.ds(row0, LANES)]
                return sq + d * d

            init = sq_v[...] if half else jnp.zeros((LANES,), jnp.float32)
            sq_v[...] = lax.fori_loop(0, NGH, group_body, init)

        pltpu.sync_copy(pred_v, pred_h.at[pl.ds(base, RW)])

        pltpu.sync_copy(sq_v, shared.at[sid])
        plsc.subcore_barrier()

        @pl.when(sid == 0)
        def _():
            pltpu.sync_copy(shared, accv)
            tot = accv[0]
            for i in range(1, NS):
                tot = tot + accv[i]
            lossv[...] = tot
            pltpu.sync_copy(lossv, loss_h.at[cid])

    return mf


def kernel(theta, X, user_indices, item_indices, ratings):
    R = user_indices.shape[0]
    ku = max(1 << (theta.shape[0] - 1).bit_length(), 4 * TC_BLK) // 4
    kx = max(1 << (X.shape[0] - 1).bit_length(), 4 * TC_BLK) // 4
    theta_q = _repack(theta, ku)
    x_q = _repack(X, kx)
    mf = _mf_sc_call(R, ku, kx)
    pred, loss_parts = mf(
        theta_q, x_q,
        user_indices.astype(jnp.int32),
        item_indices.astype(jnp.int32),
        ratings,
    )
    loss = jnp.sum(loss_parts) / R
    return pred, loss
